# Initial kernel scaffold; baseline (speedup 1.0000x reference)
import jax
import jax.numpy as jnp
from jax.experimental import pallas as pl


def kernel(input, hidden_0, hidden_1, hidden_2, hidden_3, memory, input_query, hid_query_0, hid_query_1, hid_query_2, hid_query_3, key_0, key_1, key_2, key_3, value_0, value_1, value_2, value_3):
    raise NotImplementedError("write your pallas kernel here")


# speedup vs baseline: 1.2467x; 1.2467x over previous
import jax
import jax.numpy as jnp
from jax.experimental import pallas as pl

_B, _N, _T = 64, 325, 12
_HID, _MH, _MEM, _IN, _OUT = 64, 32, 20, 2, 1
_NS = _B * _N
_S = 104
_GRID = _NS // _S
_R = _S * _T
_EPS = 1e-8


def _body(x_ref, h0_ref, h1_ref, h2_ref, h3_ref, mem_ref, iq_ref,
          hq0, hq1, hq2, hq3, k0, k1, k2, k3, v0, v1, v2, v3, out_ref):
    f32 = jnp.float32
    mem = mem_ref[:]
    xq = jnp.dot(x_ref[:], iq_ref[:], preferred_element_type=f32)
    en = jax.lax.dot_general(xq, mem, (((1,), (1,)), ((), ())),
                             preferred_element_type=f32)
    en = en - jnp.max(en, axis=-1, keepdims=True)
    p = jnp.exp(en)
    p = p / jnp.sum(p, axis=-1, keepdims=True)
    mems = jnp.dot(p, mem, preferred_element_type=f32)
    na = jnp.maximum(jnp.sqrt(jnp.sum(mems * mems, axis=-1, keepdims=True)),
                     _EPS)
    cols = []
    for h_ref, hq, kk, vv in ((h0_ref, hq0, k0, v0), (h1_ref, hq1, k1, v1),
                              (h2_ref, hq2, k2, v2), (h3_ref, hq3, k3, v3)):
        h = h_ref[:]
        q = jnp.dot(h, hq[:], preferred_element_type=f32).reshape(_S, _T, _MH)
        k = jnp.dot(h, kk[:], preferred_element_type=f32).reshape(_S, _T, _MH)
        v = jnp.dot(h, vv[:], preferred_element_type=f32).reshape(_S, _T, _MH)
        e = jax.lax.dot_general(q, k, (((2,), (2,)), ((0,), (0,))),
                                preferred_element_type=f32)
        e = e - jnp.max(e, axis=-1, keepdims=True)
        pe = jnp.exp(e)
        pe = pe / jnp.sum(pe, axis=-1, keepdims=True)
        a = jax.lax.dot_general(pe, v, (((2,), (1,)), ((0,), (0,))),
                                preferred_element_type=f32)
        a = a.reshape(_R, _MH)
        nb = jnp.maximum(jnp.sqrt(jnp.sum(a * a, axis=-1, keepdims=True)),
                         _EPS)
        dp = jnp.sum(mems * a, axis=-1, keepdims=True)
        cols.append(dp / (na * nb))
    out_ref[:] = jnp.concatenate(cols, axis=-1)


def kernel(input, hidden_0, hidden_1, hidden_2, hidden_3, memory, input_query,
           hid_query_0, hid_query_1, hid_query_2, hid_query_3,
           key_0, key_1, key_2, key_3,
           value_0, value_1, value_2, value_3):
    x = input.reshape(_NS * _T, _IN)
    hs = [h.reshape(_NS * _T, _HID)
          for h in (hidden_0, hidden_1, hidden_2, hidden_3)]

    def _full(a):
        return pl.BlockSpec(a.shape, lambda i: (0,) * a.ndim)

    row_specs = [pl.BlockSpec((_R, _IN), lambda i: (i, 0))] + \
                [pl.BlockSpec((_R, _HID), lambda i: (i, 0))] * 4
    w_args = (memory, input_query,
              hid_query_0, hid_query_1, hid_query_2, hid_query_3,
              key_0, key_1, key_2, key_3,
              value_0, value_1, value_2, value_3)
    out = pl.pallas_call(
        _body,
        grid=(_GRID,),
        in_specs=row_specs + [_full(a) for a in w_args],
        out_specs=pl.BlockSpec((_R, 4), lambda i: (i, 0)),
        out_shape=jax.ShapeDtypeStruct((_NS * _T, 4), jnp.float32),
    )(x, *hs, *w_args)
    return out.reshape(_B, _N, _T, _OUT, 4)
